# baseline (speedup 1.0000x reference)
import functools

import jax
import jax.numpy as jnp
from jax import lax
from jax.experimental import pallas as pl
from jax.experimental.pallas import tpu as pltpu
from jax.experimental.pallas import tpu_sc as plsc

N_NODES = 10000
D = 128
N_EDGES = 320000

NC = 2
NS = 16
NW = NC * NS

CHUNK = 128
NCHUNK = 79
EPW = NCHUNK * CHUNK
NPAD = 10240
RPT = NPAD // NS
LANES = 16


def _sc_aggregate(x, srcs, tgts):
    mesh = plsc.VectorSubcoreMesh(
        core_axis_name="c", subcore_axis_name="s", num_cores=NC,
        num_subcores=NS)

    @functools.partial(
        pl.kernel,
        out_type=(
            jax.ShapeDtypeStruct((NC, NPAD, D), jnp.float32),
            jax.ShapeDtypeStruct((NC, NPAD), jnp.float32),
        ),
        mesh=mesh,
        scratch_types=[
            pltpu.VMEM_SHARED((NPAD, D), jnp.float32),
            pltpu.VMEM_SHARED((NPAD,), jnp.float32),
            pltpu.VMEM((NCHUNK, CHUNK), jnp.int32),
            pltpu.VMEM((NCHUNK, CHUNK), jnp.int32),
            pltpu.VMEM((CHUNK, D), jnp.float32),
            pltpu.VMEM((RPT,), jnp.float32),
            pltpu.VMEM((CHUNK,), jnp.float32),
            pltpu.SemaphoreType.DMA,
        ],
    )
    def k(x_hbm, srcs_hbm, tgts_hbm, pagg_hbm, pcnt_hbm,
          agg_sh, cnt_sh, src_v, tgt_v, rows_v, zc_v, ones_v, sem):
        c = lax.axis_index("c")
        s = lax.axis_index("s")
        wid = c * NS + s
        base = s * RPT

        def zero_row(i, _):
            for kk in range(D // LANES):
                rows_v[i, pl.ds(kk * LANES, LANES)] = jnp.zeros(
                    (LANES,), jnp.float32)
            return ()
        lax.fori_loop(0, CHUNK, zero_row, ())
        for kk in range(RPT // LANES):
            zc_v[pl.ds(kk * LANES, LANES)] = jnp.zeros((LANES,), jnp.float32)
        for kk in range(CHUNK // LANES):
            ones_v[pl.ds(kk * LANES, LANES)] = jnp.ones((LANES,), jnp.float32)

        for kk in range(RPT // CHUNK):
            pltpu.sync_copy(rows_v,
                            agg_sh.at[pl.ds(base + kk * CHUNK, CHUNK), :])
        pltpu.sync_copy(zc_v, cnt_sh.at[pl.ds(base, RPT)])

        pltpu.sync_copy(srcs_hbm.at[wid], src_v)
        pltpu.sync_copy(tgts_hbm.at[wid], tgt_v)

        plsc.subcore_barrier()

        def step(j, _):
            cp = pltpu.async_copy(x_hbm.at[tgt_v.at[j]], rows_v, sem)
            pltpu.sync_copy(ones_v, cnt_sh.at[src_v.at[j]], add=True)
            cp.wait()
            pltpu.sync_copy(rows_v, agg_sh.at[src_v.at[j]], add=True)
            return ()
        lax.fori_loop(0, NCHUNK, step, ())

        plsc.subcore_barrier()

        pltpu.sync_copy(agg_sh.at[pl.ds(base, RPT), :],
                        pagg_hbm.at[c, pl.ds(base, RPT), :])
        pltpu.sync_copy(cnt_sh.at[pl.ds(base, RPT)],
                        pcnt_hbm.at[c, pl.ds(base, RPT)])

    return k(x, srcs, tgts)


BLK = 1024


def _tc_body(pa_ref, pc_ref, w_ref, b_ref, o_ref):
    a = pa_ref[0] + pa_ref[1]
    cnt = pc_ref[0] + pc_ref[1] + 1e-6
    a = a / cnt[:, None]
    o_ref[...] = lax.dot_general(
        a, w_ref[...], (((1,), (1,)), ((), ())),
        preferred_element_type=jnp.float32) + b_ref[...]


def _tc_linear(pagg, pcnt, W, b):
    return pl.pallas_call(
        _tc_body,
        grid=(NPAD // BLK,),
        in_specs=[
            pl.BlockSpec((NC, BLK, D), lambda i: (0, i, 0)),
            pl.BlockSpec((NC, BLK), lambda i: (0, i)),
            pl.BlockSpec((D, D), lambda i: (0, 0)),
            pl.BlockSpec((1, D), lambda i: (0, 0)),
        ],
        out_specs=pl.BlockSpec((BLK, D), lambda i: (i, 0)),
        out_shape=jax.ShapeDtypeStruct((NPAD, D), jnp.float32),
    )(pagg, pcnt, W, b)


def kernel(x, edge_index, W, b):
    src = edge_index[0]
    tgt = edge_index[1]
    pad = NW * EPW - N_EDGES
    src_p = jnp.concatenate(
        [src, jnp.full((pad,), N_NODES, dtype=jnp.int32)])
    tgt_p = jnp.concatenate([tgt, jnp.zeros((pad,), dtype=jnp.int32)])
    srcs = src_p.reshape(NW, NCHUNK, CHUNK)
    tgts = tgt_p.reshape(NW, NCHUNK, CHUNK)
    x2 = jnp.concatenate([x, jnp.zeros((4, D), jnp.float32), x], axis=0)
    tgts = tgts + jnp.where(
        jnp.arange(NW, dtype=jnp.int32)[:, None, None] >= NS, N_NODES + 4, 0)

    pagg, pcnt = _sc_aggregate(x2, srcs, tgts)
    out = _tc_linear(pagg, pcnt, W, b.reshape(1, D))
    return out[:N_NODES]


# speedup vs baseline: 1.0318x; 1.0318x over previous
import functools

import jax
import jax.numpy as jnp
from jax import lax
from jax.experimental import pallas as pl
from jax.experimental.pallas import tpu as pltpu
from jax.experimental.pallas import tpu_sc as plsc

N_NODES = 10000
D = 128
N_EDGES = 320000

NC = 2
NS = 16
NW = NC * NS

CHUNK = 128
NCHUNK = 79
EPW = NCHUNK * CHUNK
NPAD = 10240
RPT = NPAD // NS
LANES = 16


def _sc_aggregate(x, srcs, tgts):
    mesh = plsc.VectorSubcoreMesh(
        core_axis_name="c", subcore_axis_name="s", num_cores=NC,
        num_subcores=NS)

    @functools.partial(
        pl.kernel,
        out_type=(
            jax.ShapeDtypeStruct((NC, NPAD, D), jnp.float32),
            jax.ShapeDtypeStruct((NC, NPAD), jnp.float32),
        ),
        mesh=mesh,
        scratch_types=[
            pltpu.VMEM_SHARED((NPAD, D), jnp.float32),
            pltpu.VMEM_SHARED((NPAD,), jnp.float32),
            pltpu.VMEM((NCHUNK, CHUNK), jnp.int32),
            pltpu.VMEM((NCHUNK, CHUNK), jnp.int32),
            pltpu.VMEM((CHUNK, D), jnp.float32),
            pltpu.VMEM((RPT,), jnp.float32),
            pltpu.VMEM((CHUNK,), jnp.float32),
            pltpu.SemaphoreType.DMA,
        ],
    )
    def k(x_hbm, srcs_hbm, tgts_hbm, pagg_hbm, pcnt_hbm,
          agg_sh, cnt_sh, src_v, tgt_v, rows_v, zc_v, ones_v, sem):
        c = lax.axis_index("c")
        s = lax.axis_index("s")
        wid = c * NS + s
        base = s * RPT

        def zero_row(i, _):
            for kk in range(D // LANES):
                rows_v[i, pl.ds(kk * LANES, LANES)] = jnp.zeros(
                    (LANES,), jnp.float32)
            return ()
        lax.fori_loop(0, CHUNK, zero_row, ())
        for kk in range(RPT // LANES):
            zc_v[pl.ds(kk * LANES, LANES)] = jnp.zeros((LANES,), jnp.float32)
        for kk in range(CHUNK // LANES):
            ones_v[pl.ds(kk * LANES, LANES)] = jnp.ones((LANES,), jnp.float32)

        for kk in range(RPT // CHUNK):
            pltpu.sync_copy(rows_v,
                            agg_sh.at[pl.ds(base + kk * CHUNK, CHUNK), :])
        pltpu.sync_copy(zc_v, cnt_sh.at[pl.ds(base, RPT)])

        pltpu.sync_copy(srcs_hbm.at[wid], src_v)
        pltpu.sync_copy(tgts_hbm.at[wid], tgt_v)

        plsc.subcore_barrier()

        def step(j, _):
            cp = pltpu.async_copy(x_hbm.at[tgt_v.at[j]], rows_v, sem)
            pltpu.sync_copy(ones_v, cnt_sh.at[src_v.at[j]], add=True)
            cp.wait()
            pltpu.sync_copy(rows_v, agg_sh.at[src_v.at[j]], add=True)
            return ()
        lax.fori_loop(0, NCHUNK, step, ())

        plsc.subcore_barrier()

        pltpu.sync_copy(agg_sh.at[pl.ds(base, RPT), :],
                        pagg_hbm.at[c, pl.ds(base, RPT), :])
        pltpu.sync_copy(cnt_sh.at[pl.ds(base, RPT)],
                        pcnt_hbm.at[c, pl.ds(base, RPT)])

    return k(x, srcs, tgts)


BLK = 1024


def _tc_body(pa_ref, pc_ref, w_ref, b_ref, o_ref):
    a = pa_ref[0] + pa_ref[1]
    cnt = pc_ref[0] + pc_ref[1] + 1e-6
    a = a / cnt[:, None]
    o_ref[...] = lax.dot_general(
        a, w_ref[...], (((1,), (1,)), ((), ())),
        preferred_element_type=jnp.float32) + b_ref[...]


def _tc_linear(pagg, pcnt, W, b):
    return pl.pallas_call(
        _tc_body,
        grid=(NPAD // BLK,),
        in_specs=[
            pl.BlockSpec((NC, BLK, D), lambda i: (0, i, 0)),
            pl.BlockSpec((NC, BLK), lambda i: (0, i)),
            pl.BlockSpec((D, D), lambda i: (0, 0)),
            pl.BlockSpec((1, D), lambda i: (0, 0)),
        ],
        out_specs=pl.BlockSpec((BLK, D), lambda i: (i, 0)),
        out_shape=jax.ShapeDtypeStruct((NPAD, D), jnp.float32),
    )(pagg, pcnt, W, b)


def kernel(x, edge_index, W, b):
    src = edge_index[0]
    tgt = edge_index[1]
    pad = NW * EPW - N_EDGES
    src_p = jnp.concatenate(
        [src, jnp.full((pad,), N_NODES, dtype=jnp.int32)])
    tgt_p = jnp.concatenate([tgt, jnp.zeros((pad,), dtype=jnp.int32)])
    srcs = src_p.reshape(NW, NCHUNK, CHUNK)
    tgts = tgt_p.reshape(NW, NCHUNK, CHUNK)
    x2 = jnp.concatenate([x, x], axis=0)
    tgts = tgts + jnp.where(
        jnp.arange(NW, dtype=jnp.int32)[:, None, None] >= NS, N_NODES, 0)

    pagg, pcnt = _sc_aggregate(x2, srcs, tgts)
    out = _tc_linear(pagg, pcnt, W, b.reshape(1, D))
    return out[:N_NODES]
